# baseline (speedup 1.0000x reference)
import functools

import jax
import jax.numpy as jnp
from jax import lax
from jax.experimental import pallas as pl
from jax.experimental.pallas import tpu as pltpu
from jax.experimental.pallas import tpu_sc as plsc

MAXP = 512
ED = 32
SQ = 2048
SK = 2048
NW = 32
RPW = SQ // NW
GRP = SK * ED // 128
WIN2 = GRP + RPW
GPP = 1024
T4R = 4 * GPP * 4
T4W = T4R // NW
GCH = 128


def _build_body(delta_hbm, table_hbm, t4_hbm, d_v, idx_v, g_v, sem):
    w = lax.axis_index("s") * 2 + lax.axis_index("c")
    p = w // 8
    j0 = (w % 8) * T4W

    pltpu.sync_copy(delta_hbm, d_v)
    dvec = d_v[...]
    lane = lax.iota(jnp.int32, 16)

    def ibody(t, c):
        v = lane + (j0 + t * 16 + p - (SK - 1)) + dvec
        v = jnp.minimum(jnp.maximum(v, -MAXP), MAXP) + MAXP
        idx_v[pl.ds(t * 16, 16)] = v
        return c
    lax.fori_loop(0, T4W // 16, ibody, 0)

    def gbody(c, x):
        pltpu.async_copy(
            table_hbm.at[idx_v.at[pl.ds(c * GCH, GCH)]],
            g_v.at[pl.ds(c * GCH, GCH)], sem)
        return x
    lax.fori_loop(0, T4W // GCH, gbody, 0)

    def gwait(c, x):
        pltpu.make_async_copy(
            table_hbm.at[idx_v.at[pl.ds(0, GCH)]],
            g_v.at[pl.ds(0, GCH)], sem).wait()
        return x
    lax.fori_loop(0, T4W // GCH, gwait, 0)

    pltpu.sync_copy(g_v, t4_hbm.at[pl.ds(w * T4W, T4W)])


def _mat_body(t4g_hbm, out_hbm, win2, sem):
    w = lax.axis_index("s") * 2 + lax.axis_index("c")
    c0 = w % 4
    g = w // 4
    p = 3 - c0
    gs63 = 448 - 64 * g

    pltpu.sync_copy(t4g_hbm.at[pl.ds(p * GPP + gs63, WIN2)], win2)

    base = c0 + 256 * g
    def sbody(r, x):
        pltpu.async_copy(
            win2.at[pl.ds(RPW - 1 - r, GRP)],
            out_hbm.at[base + 4 * r], sem)
        return x
    lax.fori_loop(0, RPW, sbody, 0)

    def swait(r, x):
        pltpu.make_async_copy(
            win2.at[pl.ds(0, GRP)],
            out_hbm.at[base], sem).wait()
        return x
    lax.fori_loop(0, RPW, swait, 0)


def kernel(seq_len_q, seq_len_k, embeddings_table):
    delta = jnp.full((16,), jnp.int32(seq_len_k) - jnp.int32(seq_len_q),
                     dtype=jnp.int32)
    mesh = plsc.VectorSubcoreMesh(core_axis_name="c", subcore_axis_name="s")
    params = pltpu.CompilerParams(use_tc_tiling_on_sc=False)

    build = functools.partial(
        pl.kernel, mesh=mesh,
        out_type=jax.ShapeDtypeStruct((T4R, ED), jnp.float32),
        scratch_types=[
            pltpu.VMEM((16,), jnp.int32),
            pltpu.VMEM((T4W,), jnp.int32),
            pltpu.VMEM((T4W, ED), jnp.float32),
            pltpu.SemaphoreType.DMA,
        ],
        compiler_params=params,
    )(_build_body)
    t4 = build(delta, embeddings_table)

    mat = functools.partial(
        pl.kernel, mesh=mesh,
        out_type=jax.ShapeDtypeStruct((SQ, GRP, 128), jnp.float32),
        scratch_types=[
            pltpu.VMEM((WIN2, 128), jnp.float32),
            pltpu.SemaphoreType.DMA,
        ],
        compiler_params=params,
    )(_mat_body)
    out = mat(t4.reshape(T4R * ED // 128, 128))
    return out.reshape(SQ, SK, ED)
